# baseline (speedup 1.0000x reference)
import functools

import jax
import jax.numpy as jnp
from jax import lax
from jax.experimental import pallas as pl
from jax.experimental.pallas import tpu as pltpu
from jax.experimental.pallas import tpu_sc as plsc

L = 16
GB = 128
K = 3200
CAP = 16384
NEG_INF = float("-inf")


@functools.lru_cache(maxsize=None)
def _build_sc_agg(N, E, C, NW):
    ROWS = -(-N // NW)
    NPAD = ROWS * NW
    NCH = -(-E // K)
    assert C % L == 0 and (ROWS * C) % L == 0 and K % (2 * L) == 0
    CB = C // L
    mesh = plsc.VectorSubcoreMesh(core_axis_name="c", subcore_axis_name="s")
    info = plsc.get_sparse_core_info()
    NC = info.num_cores

    def body(x1_hbm, src_hbm, dst_hbm, agg_hbm,
             agg_v, dst_a, dst_b, src_a, src_b, srcc, dstc,
             rows_a, rows_b, sem_ca, sem_cb, sem_ga, sem_gb):
        wid = lax.axis_index("s") * NC + lax.axis_index("c")
        lo = wid * ROWS
        hi = lo + ROWS

        ninf = jnp.full((L,), NEG_INF, dtype=jnp.float32)
        zero = jnp.zeros((L,), dtype=jnp.int32)
        dummy = jnp.full((L,), ROWS * C, dtype=jnp.int32)

        def init_agg(r, _):
            agg_v[pl.ds(r * L, L)] = ninf
            return 0
        lax.fori_loop(0, ROWS * C // L, init_agg, 0)

        def init_srcc(r, _):
            srcc[pl.ds(r * L, L)] = zero
            return 0
        lax.fori_loop(0, (CAP + GB) // L, init_srcc, 0)

        def fire_chunk(i, dref, sref, sem):
            pltpu.async_copy(dst_hbm.at[pl.ds(i * K, K)], dref, sem)
            pltpu.async_copy(src_hbm.at[pl.ds(i * K, K)], sref, sem)

        def wait_chunk(i, dref, sref, sem):
            pltpu.make_async_copy(dst_hbm.at[pl.ds(i * K, K)], dref, sem).wait()
            pltpu.make_async_copy(src_hbm.at[pl.ds(i * K, K)], sref, sem).wait()

        last = jnp.full((L,), L - 1, jnp.int32)

        def lane_bcast(v):
            return v.at[last].get(mode="promise_in_bounds")

        def scan_chunk(dref, sref, cntv0):
            UN = 4

            def scan_body(j, cntv):
                ds_ = [dref[pl.ds((j * UN + u) * L, L)] for u in range(UN)]
                ss = [sref[pl.ds((j * UN + u) * L, L)] for u in range(UN)]
                ms = [(d >= lo) & (d < hi) for d in ds_]
                pcs = [plsc.cumsum(m.astype(jnp.int32)) for m in ms]
                for u in range(UN):
                    idx = cntv + pcs[u] - 1
                    plsc.store_scatter(srcc, [idx], ss[u], mask=ms[u])
                    plsc.store_scatter(dstc, [idx], (ds_[u] - lo) * C,
                                       mask=ms[u])
                    cntv = cntv + lane_bcast(pcs[u])
                return cntv
            return lax.fori_loop(0, K // (UN * L), scan_body, cntv0)

        def fire(g, rows, sem):
            pltpu.async_copy(
                x1_hbm.at[srcc.at[pl.ds(g * GB, GB)]], rows, sem
            )

        def wait(g, rows, sem):
            pltpu.make_async_copy(
                x1_hbm.at[srcc.at[pl.ds(g * GB, GB)]], rows, sem
            ).wait()

        def accum(rows, g):
            def blk(b, _):
                dv = dstc[pl.ds(g * GB + b * L, L)]
                offs = [dv[lane] for lane in range(L)]
                for lane in range(L):
                    off = offs[lane]
                    e = b * L + lane
                    avs = [agg_v[pl.ds(off + c * L, L)] for c in range(CB)]
                    rvs = [rows[e, pl.ds(c * L, L)] for c in range(CB)]
                    for c in range(CB):
                        agg_v[pl.ds(off + c * L, L)] = jnp.maximum(
                            avs[c], rvs[c])
                return 0
            lax.fori_loop(0, GB // L, blk, 0)

        def drain_from(gd, cnt):
            for t in range(GB // L):
                dstc[pl.ds(cnt + t * L, L)] = dummy
            ngr = (cnt + GB - 1) // GB

            @pl.when(gd < ngr)
            def _():
                fire(gd, rows_a, sem_ga)

            def pair_body(p, _):
                g0 = gd + 2 * p
                g1 = g0 + 1

                @pl.when(g1 < ngr)
                def _():
                    fire(g1, rows_b, sem_gb)
                wait(g0, rows_a, sem_ga)
                accum(rows_a, g0)

                @pl.when(g1 + 1 < ngr)
                def _():
                    fire(g1 + 1, rows_a, sem_ga)

                @pl.when(g1 < ngr)
                def _():
                    wait(g1, rows_b, sem_gb)
                    accum(rows_b, g1)
                return 0
            lax.fori_loop(0, (ngr - gd + 1) // 2, pair_body, 0)

        def acc_pair(gd):
            wait(gd, rows_a, sem_ga)
            accum(rows_a, gd)
            wait(gd + 1, rows_b, sem_gb)
            accum(rows_b, gd + 1)
            return gd + 2

        fire_chunk(0, dst_a, src_a, sem_ca)

        def cpair_body(p, carry):
            cntv, gdone, infl = carry
            i0 = 2 * p
            i1 = i0 + 1

            @pl.when(i1 < NCH)
            def _():
                fire_chunk(i1, dst_b, src_b, sem_cb)
            wait_chunk(i0, dst_a, src_a, sem_ca)
            cntv = scan_chunk(dst_a, src_a, cntv)

            @pl.when(i1 + 1 < NCH)
            def _():
                fire_chunk(i1 + 1, dst_a, src_a, sem_ca)

            def second():
                wait_chunk(i1, dst_b, src_b, sem_cb)
                return scan_chunk(dst_b, src_b, cntv)
            cntv = lax.cond(i1 < NCH, second, lambda: cntv)

            gdone = lax.cond(infl == 1, lambda: acc_pair(gdone),
                             lambda: gdone)
            cnt = cntv[0]

            def _dr():
                drain_from(gdone, cnt)
                return (jnp.zeros((L,), jnp.int32), jnp.int32(0),
                        jnp.int32(0))

            def _fire():
                canf = (cnt // GB - gdone) >= 2

                @pl.when(canf)
                def _():
                    fire(gdone, rows_a, sem_ga)
                    fire(gdone + 1, rows_b, sem_gb)
                return (cntv, gdone, canf.astype(jnp.int32))
            return lax.cond(cnt > CAP - 2 * K, _dr, _fire)
        cntv, gdone, infl = lax.fori_loop(
            0, (NCH + 1) // 2, cpair_body,
            (jnp.zeros((L,), jnp.int32), jnp.int32(0), jnp.int32(0)))
        gdone = lax.cond(infl == 1, lambda: acc_pair(gdone), lambda: gdone)
        drain_from(gdone, cntv[0])

        pltpu.sync_copy(agg_v.at[pl.ds(0, ROWS * C)],
                        agg_hbm.at[pl.ds(lo * C, ROWS * C)])

    return pl.kernel(
        body,
        out_type=jax.ShapeDtypeStruct((NPAD * C,), jnp.float32),
        mesh=mesh,
        scratch_types=[
            pltpu.VMEM(((ROWS + 1) * C,), jnp.float32),
            pltpu.VMEM((K,), jnp.int32),
            pltpu.VMEM((K,), jnp.int32),
            pltpu.VMEM((K,), jnp.int32),
            pltpu.VMEM((K,), jnp.int32),
            pltpu.VMEM((CAP + GB,), jnp.int32),
            pltpu.VMEM((CAP + GB,), jnp.int32),
            pltpu.VMEM((GB, C), jnp.float32),
            pltpu.VMEM((GB, C), jnp.float32),
            pltpu.SemaphoreType.DMA,
            pltpu.SemaphoreType.DMA,
            pltpu.SemaphoreType.DMA,
            pltpu.SemaphoreType.DMA,
        ],
        compiler_params=pltpu.CompilerParams(needs_layout_passes=False),
    ), NPAD


def _mlp_body(x_ref, a_ref, w1_ref, b1_ref, w2_ref, b2_ref, o_ref):
    a = a_ref[...]
    a = jnp.where(a == NEG_INF, 0.0, a)
    h = x_ref[...] + a
    h = jnp.dot(h, w1_ref[...], preferred_element_type=jnp.float32)
    h = jnp.maximum(h + b1_ref[...], 0.0)
    o = jnp.dot(h, w2_ref[...], preferred_element_type=jnp.float32)
    o_ref[...] = o + b2_ref[...]


@functools.lru_cache(maxsize=None)
def _build_mlp(N, C, BR):
    grid = (N // BR,)
    return pl.pallas_call(
        _mlp_body,
        grid=grid,
        in_specs=[
            pl.BlockSpec((BR, C), lambda i: (i, 0)),
            pl.BlockSpec((BR, C), lambda i: (i, 0)),
            pl.BlockSpec((C, C), lambda i: (0, 0)),
            pl.BlockSpec((1, C), lambda i: (0, 0)),
            pl.BlockSpec((C, C), lambda i: (0, 0)),
            pl.BlockSpec((1, C), lambda i: (0, 0)),
        ],
        out_specs=pl.BlockSpec((BR, C), lambda i: (i, 0)),
        out_shape=jax.ShapeDtypeStruct((N, C), jnp.float32),
    )


@jax.jit
def kernel(x1, adj, W1, b1, W2, b2):
    N, C = x1.shape
    E = adj.shape[1]
    NW = 32
    sc_agg, NPAD = _build_sc_agg(N, E, C, NW)
    src = adj[0]
    dst = adj[1]
    EPAD = -(-E // K) * K
    if EPAD != E:
        src = jnp.concatenate([src, jnp.zeros((EPAD - E,), jnp.int32)])
        dst = jnp.concatenate([dst, jnp.full((EPAD - E,), NPAD, jnp.int32)])
    agg = sc_agg(x1, src, dst).reshape(NPAD, C)[:N]
    BR = 1000 if N % 1000 == 0 else 8
    mlp = _build_mlp(N, C, BR)
    return mlp(x1, agg, W1, b1.reshape(1, C), W2, b2.reshape(1, C))


# speedup vs baseline: 1.1420x; 1.1420x over previous
import functools

import jax
import jax.numpy as jnp
from jax import lax
from jax.experimental import pallas as pl
from jax.experimental.pallas import tpu as pltpu
from jax.experimental.pallas import tpu_sc as plsc

L = 16
GB = 128
K = 3200
CAP = 16384
NEG_INF = float("-inf")


@functools.lru_cache(maxsize=None)
def _build_sc_agg(N, E, C, NW):
    ROWS = -(-N // NW)
    NPAD = ROWS * NW
    NCH = -(-E // K)
    assert C % L == 0 and (ROWS * C) % L == 0 and K % (2 * L) == 0
    CB = C // L
    mesh = plsc.VectorSubcoreMesh(core_axis_name="c", subcore_axis_name="s")
    info = plsc.get_sparse_core_info()
    NC = info.num_cores

    def body(x1_hbm, src_hbm, dst_hbm, agg_hbm,
             agg_v, dst_a, dst_b, src_a, src_b, srcc, dstc,
             rows_a, rows_b, sem_ca, sem_cb, sem_ga, sem_gb):
        wid = lax.axis_index("s") * NC + lax.axis_index("c")
        lo = wid * ROWS
        hi = lo + ROWS

        ninf = jnp.full((L,), NEG_INF, dtype=jnp.float32)
        zero = jnp.zeros((L,), dtype=jnp.int32)
        dummy = jnp.full((L,), ROWS * C, dtype=jnp.int32)

        def init_agg(r, _):
            agg_v[pl.ds(r * L, L)] = ninf
            return 0
        lax.fori_loop(0, ROWS * C // L, init_agg, 0)

        def init_srcc(r, _):
            srcc[pl.ds(r * L, L)] = zero
            return 0
        lax.fori_loop(0, (CAP + GB) // L, init_srcc, 0)

        def fire_chunk(i, dref, sref, sem):
            pltpu.async_copy(dst_hbm.at[pl.ds(i * K, K)], dref, sem)
            pltpu.async_copy(src_hbm.at[pl.ds(i * K, K)], sref, sem)

        def wait_chunk(i, dref, sref, sem):
            pltpu.make_async_copy(dst_hbm.at[pl.ds(i * K, K)], dref, sem).wait()
            pltpu.make_async_copy(src_hbm.at[pl.ds(i * K, K)], sref, sem).wait()

        last = jnp.full((L,), L - 1, jnp.int32)

        def lane_bcast(v):
            return v.at[last].get(mode="promise_in_bounds")

        def scan_chunk(dref, sref, cntv0):
            UN = 8

            def scan_body(j, cntv):
                ds_ = [dref[pl.ds((j * UN + u) * L, L)] for u in range(UN)]
                ss = [sref[pl.ds((j * UN + u) * L, L)] for u in range(UN)]
                ms = [(d >= lo) & (d < hi) for d in ds_]
                pcs = [plsc.cumsum(m.astype(jnp.int32)) for m in ms]
                for u in range(UN):
                    idx = cntv + pcs[u] - 1
                    plsc.store_scatter(srcc, [idx], ss[u], mask=ms[u])
                    plsc.store_scatter(dstc, [idx], (ds_[u] - lo) * C,
                                       mask=ms[u])
                    cntv = cntv + lane_bcast(pcs[u])
                return cntv
            return lax.fori_loop(0, K // (UN * L), scan_body, cntv0)

        def fire(g, rows, sem):
            pltpu.async_copy(
                x1_hbm.at[srcc.at[pl.ds(g * GB, GB)]], rows, sem
            )

        def wait(g, rows, sem):
            pltpu.make_async_copy(
                x1_hbm.at[srcc.at[pl.ds(g * GB, GB)]], rows, sem
            ).wait()

        def accum(rows, g):
            def blk(b, _):
                dv = dstc[pl.ds(g * GB + b * L, L)]
                offs = [dv[lane] for lane in range(L)]
                for lane in range(L):
                    off = offs[lane]
                    e = b * L + lane
                    avs = [agg_v[pl.ds(off + c * L, L)] for c in range(CB)]
                    rvs = [rows[e, pl.ds(c * L, L)] for c in range(CB)]
                    for c in range(CB):
                        agg_v[pl.ds(off + c * L, L)] = jnp.maximum(
                            avs[c], rvs[c])
                return 0
            lax.fori_loop(0, GB // L, blk, 0)

        def drain(cnt):
            for t in range(GB // L):
                dstc[pl.ds(cnt + t * L, L)] = dummy
            ngr = (cnt + GB - 1) // GB

            @pl.when(ngr > 0)
            def _():
                fire(0, rows_a, sem_ga)

            def pair_body(p, _):
                g0 = 2 * p
                g1 = g0 + 1

                @pl.when(g1 < ngr)
                def _():
                    fire(g1, rows_b, sem_gb)
                wait(g0, rows_a, sem_ga)
                accum(rows_a, g0)

                @pl.when(g1 + 1 < ngr)
                def _():
                    fire(g1 + 1, rows_a, sem_ga)

                @pl.when(g1 < ngr)
                def _():
                    wait(g1, rows_b, sem_gb)
                    accum(rows_b, g1)
                return 0
            lax.fori_loop(0, (ngr + 1) // 2, pair_body, 0)

        fire_chunk(0, dst_a, src_a, sem_ca)

        def cpair_body(p, cntv):
            i0 = 2 * p
            i1 = i0 + 1

            @pl.when(i1 < NCH)
            def _():
                fire_chunk(i1, dst_b, src_b, sem_cb)
            wait_chunk(i0, dst_a, src_a, sem_ca)
            cntv = scan_chunk(dst_a, src_a, cntv)

            @pl.when(i1 + 1 < NCH)
            def _():
                fire_chunk(i1 + 1, dst_a, src_a, sem_ca)

            def second():
                wait_chunk(i1, dst_b, src_b, sem_cb)
                return scan_chunk(dst_b, src_b, cntv)
            cntv = lax.cond(i1 < NCH, second, lambda: cntv)
            cnt = cntv[0]

            def _dr():
                drain(cnt)
                return jnp.zeros((L,), jnp.int32)
            cntv = lax.cond(cnt > CAP - 2 * K, _dr, lambda: cntv)
            return cntv
        cntv = lax.fori_loop(0, (NCH + 1) // 2, cpair_body,
                             jnp.zeros((L,), jnp.int32))
        drain(cntv[0])

        pltpu.sync_copy(agg_v.at[pl.ds(0, ROWS * C)],
                        agg_hbm.at[pl.ds(lo * C, ROWS * C)])

    return pl.kernel(
        body,
        out_type=jax.ShapeDtypeStruct((NPAD * C,), jnp.float32),
        mesh=mesh,
        scratch_types=[
            pltpu.VMEM(((ROWS + 1) * C,), jnp.float32),
            pltpu.VMEM((K,), jnp.int32),
            pltpu.VMEM((K,), jnp.int32),
            pltpu.VMEM((K,), jnp.int32),
            pltpu.VMEM((K,), jnp.int32),
            pltpu.VMEM((CAP + GB,), jnp.int32),
            pltpu.VMEM((CAP + GB,), jnp.int32),
            pltpu.VMEM((GB, C), jnp.float32),
            pltpu.VMEM((GB, C), jnp.float32),
            pltpu.SemaphoreType.DMA,
            pltpu.SemaphoreType.DMA,
            pltpu.SemaphoreType.DMA,
            pltpu.SemaphoreType.DMA,
        ],
        compiler_params=pltpu.CompilerParams(needs_layout_passes=False),
    ), NPAD


def _mlp_body(x_ref, a_ref, w1_ref, b1_ref, w2_ref, b2_ref, o_ref):
    a = a_ref[...]
    a = jnp.where(a == NEG_INF, 0.0, a)
    h = x_ref[...] + a
    h = jnp.dot(h, w1_ref[...], preferred_element_type=jnp.float32)
    h = jnp.maximum(h + b1_ref[...], 0.0)
    o = jnp.dot(h, w2_ref[...], preferred_element_type=jnp.float32)
    o_ref[...] = o + b2_ref[...]


@functools.lru_cache(maxsize=None)
def _build_mlp(N, C, BR):
    grid = (N // BR,)
    return pl.pallas_call(
        _mlp_body,
        grid=grid,
        in_specs=[
            pl.BlockSpec((BR, C), lambda i: (i, 0)),
            pl.BlockSpec((BR, C), lambda i: (i, 0)),
            pl.BlockSpec((C, C), lambda i: (0, 0)),
            pl.BlockSpec((1, C), lambda i: (0, 0)),
            pl.BlockSpec((C, C), lambda i: (0, 0)),
            pl.BlockSpec((1, C), lambda i: (0, 0)),
        ],
        out_specs=pl.BlockSpec((BR, C), lambda i: (i, 0)),
        out_shape=jax.ShapeDtypeStruct((N, C), jnp.float32),
    )


@jax.jit
def kernel(x1, adj, W1, b1, W2, b2):
    N, C = x1.shape
    E = adj.shape[1]
    NW = 32
    sc_agg, NPAD = _build_sc_agg(N, E, C, NW)
    src = adj[0]
    dst = adj[1]
    EPAD = -(-E // K) * K
    if EPAD != E:
        src = jnp.concatenate([src, jnp.zeros((EPAD - E,), jnp.int32)])
        dst = jnp.concatenate([dst, jnp.full((EPAD - E,), NPAD, jnp.int32)])
    agg = sc_agg(x1, src, dst).reshape(NPAD, C)[:N]
    BR = 1000 if N % 1000 == 0 else 8
    mlp = _build_mlp(N, C, BR)
    return mlp(x1, agg, W1, b1.reshape(1, C), W2, b2.reshape(1, C))
